# Initial kernel scaffold; baseline (speedup 1.0000x reference)
import jax
import jax.numpy as jnp
from jax.experimental import pallas as pl


def kernel(ids, pe):
    raise NotImplementedError("write your pallas kernel here")


# speedup vs baseline: 2.3799x; 2.3799x over previous
import functools

import jax
import jax.numpy as jnp
from jax import lax
from jax.experimental import pallas as pl
from jax.experimental.pallas import tpu as pltpu
from jax.experimental.pallas import tpu_sc as plsc

D_MODEL = 1024
NUM_CORES = 2
NUM_SUBCORES = 16
NUM_WORKERS = NUM_CORES * NUM_SUBCORES
CHUNK = 32


@functools.partial(jax.jit, static_argnames=())
def _gather_rows(ids2, pe):
    nchunk = ids2.shape[1]
    b_per_w = nchunk * CHUNK
    total = NUM_WORKERS * b_per_w
    mesh = plsc.VectorSubcoreMesh(
        core_axis_name="c", subcore_axis_name="s",
        num_cores=NUM_CORES, num_subcores=NUM_SUBCORES)

    @functools.partial(
        pl.kernel,
        mesh=mesh,
        out_type=jax.ShapeDtypeStruct((total, D_MODEL), jnp.float32),
        scratch_types=[
            pltpu.VMEM((nchunk, CHUNK), jnp.int32),
            pltpu.VMEM((CHUNK, D_MODEL), jnp.float32),
            pltpu.VMEM((CHUNK, D_MODEL), jnp.float32),
            pltpu.SemaphoreType.DMA,
            pltpu.SemaphoreType.DMA,
        ],
    )
    def k(ids_hbm, pe_hbm, out_hbm, idx_v, buf0, buf1, sem0, sem1):
        bufs = (buf0, buf1)
        sems = (sem0, sem1)
        wid = lax.axis_index("s") * NUM_CORES + lax.axis_index("c")
        base = wid * b_per_w
        pltpu.sync_copy(ids_hbm.at[wid], idx_v)
        pltpu.async_copy(pe_hbm.at[idx_v.at[0]], buf0, sem0)
        pltpu.async_copy(pe_hbm.at[idx_v.at[1]], buf1, sem1)

        @pl.loop(0, nchunk - 2, step=2)
        def _(g):
            for b in range(2):
                chunk = g + b
                pltpu.make_async_copy(
                    pe_hbm.at[idx_v.at[chunk]], bufs[b], sems[b]).wait()
                pltpu.sync_copy(
                    bufs[b], out_hbm.at[pl.ds(base + chunk * CHUNK, CHUNK)])
                pltpu.async_copy(
                    pe_hbm.at[idx_v.at[chunk + 2]], bufs[b], sems[b])

        for b in range(2):
            chunk = nchunk - 2 + b
            pltpu.make_async_copy(
                pe_hbm.at[idx_v.at[chunk]], bufs[b], sems[b]).wait()
            pltpu.sync_copy(
                bufs[b], out_hbm.at[pl.ds(base + chunk * CHUNK, CHUNK)])

    return k(ids2, pe)


def kernel(ids, pe):
    n, s = ids.shape
    total = n * s
    b_per_w = total // NUM_WORKERS
    nchunk = b_per_w // CHUNK
    ids2 = ids.reshape(NUM_WORKERS, nchunk, CHUNK).astype(jnp.int32)
    out = _gather_rows(ids2, pe)
    return out.reshape(n, s, D_MODEL)


# speedup vs baseline: 2.3826x; 1.0011x over previous
import functools

import jax
import jax.numpy as jnp
from jax import lax
from jax.experimental import pallas as pl
from jax.experimental.pallas import tpu as pltpu
from jax.experimental.pallas import tpu_sc as plsc

D_MODEL = 1024
NUM_CORES = 2
NUM_SUBCORES = 16
NUM_WORKERS = NUM_CORES * NUM_SUBCORES
CHUNK = 16
NBUF = 4


@functools.partial(jax.jit, static_argnames=())
def _gather_rows(ids2, pe):
    nchunk = ids2.shape[1]
    b_per_w = nchunk * CHUNK
    total = NUM_WORKERS * b_per_w
    mesh = plsc.VectorSubcoreMesh(
        core_axis_name="c", subcore_axis_name="s",
        num_cores=NUM_CORES, num_subcores=NUM_SUBCORES)

    @functools.partial(
        pl.kernel,
        mesh=mesh,
        out_type=jax.ShapeDtypeStruct((total, D_MODEL), jnp.float32),
        scratch_types=[
            pltpu.VMEM((nchunk, CHUNK), jnp.int32),
        ] + [pltpu.VMEM((CHUNK, D_MODEL), jnp.float32)] * NBUF
          + [pltpu.SemaphoreType.DMA] * NBUF,
    )
    def k(ids_hbm, pe_hbm, out_hbm, idx_v, *bufs_sems):
        bufs = bufs_sems[:NBUF]
        sems = bufs_sems[NBUF:]
        wid = lax.axis_index("s") * NUM_CORES + lax.axis_index("c")
        base = wid * b_per_w
        pltpu.sync_copy(ids_hbm.at[wid], idx_v)
        for b in range(NBUF):
            pltpu.async_copy(pe_hbm.at[idx_v.at[b]], bufs[b], sems[b])

        @pl.loop(0, nchunk - NBUF, step=NBUF)
        def _(g):
            for b in range(NBUF):
                chunk = g + b
                pltpu.make_async_copy(
                    pe_hbm.at[idx_v.at[chunk]], bufs[b], sems[b]).wait()
                pltpu.sync_copy(
                    bufs[b], out_hbm.at[pl.ds(base + chunk * CHUNK, CHUNK)])
                pltpu.async_copy(
                    pe_hbm.at[idx_v.at[chunk + NBUF]], bufs[b], sems[b])

        for b in range(NBUF):
            chunk = nchunk - NBUF + b
            pltpu.make_async_copy(
                pe_hbm.at[idx_v.at[chunk]], bufs[b], sems[b]).wait()
            pltpu.sync_copy(
                bufs[b], out_hbm.at[pl.ds(base + chunk * CHUNK, CHUNK)])

    return k(ids2, pe)


def kernel(ids, pe):
    n, s = ids.shape
    total = n * s
    b_per_w = total // NUM_WORKERS
    nchunk = b_per_w // CHUNK
    ids2 = ids.reshape(NUM_WORKERS, nchunk, CHUNK).astype(jnp.int32)
    out = _gather_rows(ids2, pe)
    return out.reshape(n, s, D_MODEL)
